# baseline (speedup 1.0000x reference)
import functools

import jax
import jax.numpy as jnp
from jax import lax
from jax.experimental import pallas as pl
from jax.experimental.pallas import tpu as pltpu
from jax.experimental.pallas import tpu_sc as plsc

EMB_DIM = 64
PART_DIM = 128

NUM_SC_CORES = 2
NUM_SC_SUBCORES = 16
NUM_WORKERS = NUM_SC_CORES * NUM_SC_SUBCORES
LANES = 16

def _pack_halves(acc):
    bits = lax.bitcast_convert_type(
        acc.astype(jnp.bfloat16).astype(jnp.float32), jnp.uint32)
    lo = bits[:, :EMB_DIM] >> 16
    hi = bits[:, EMB_DIM:] & jnp.uint32(0xFFFF0000)
    return lo | hi


def _tc_project_packed(table_t, w1, w2, b2d, blk):
    vocab = table_t.shape[1]
    nb = (vocab + blk - 1) // blk

    def proj_kernel(t_ref, w1_ref, w2_ref, b_ref, o_ref):
        tb = t_ref[...]
        acc1 = lax.dot_general(
            tb, w1_ref[...], (((0,), (1,)), ((), ())),
            preferred_element_type=jnp.float32,
        ) + b_ref[...]
        acc2 = lax.dot_general(
            tb, w2_ref[...], (((0,), (1,)), ((), ())),
            preferred_element_type=jnp.float32,
        )
        o_ref[...] = jnp.concatenate(
            [_pack_halves(acc1), _pack_halves(acc2)], axis=1)

    return pl.pallas_call(
        proj_kernel,
        grid=(nb,),
        in_specs=[
            pl.BlockSpec((EMB_DIM, blk), lambda i: (0, i)),
            pl.BlockSpec((PART_DIM, EMB_DIM), lambda i: (0, 0)),
            pl.BlockSpec((PART_DIM, EMB_DIM), lambda i: (0, 0)),
            pl.BlockSpec((1, PART_DIM), lambda i: (0, 0)),
        ],
        out_specs=pl.BlockSpec((blk, PART_DIM), lambda i: (i, 0)),
        out_shape=jax.ShapeDtypeStruct((vocab, PART_DIM), jnp.uint32),
    )(table_t, w1, w2, b2d)


def _sc_gather_combine(idx_a2, idx_m2, t12, chunk):
    n = idx_a2.shape[0]
    per_w = n // NUM_WORKERS
    n_chunks = per_w // chunk
    assert per_w % chunk == 0 and chunk % 8 == 0 and n_chunks % 2 == 0

    mesh = plsc.VectorSubcoreMesh(core_axis_name="c", subcore_axis_name="s")

    row_buf = lambda: pltpu.VMEM((chunk, EMB_DIM), jnp.uint32)
    out_buf = lambda: pltpu.VMEM((chunk, PART_DIM), jnp.float32)
    idx_buf = lambda: pltpu.VMEM((chunk,), jnp.int32)

    @functools.partial(
        pl.kernel,
        mesh=mesh,
        out_type=jax.ShapeDtypeStruct((n, PART_DIM), jnp.float32),
        scratch_types=[
            [idx_buf(), idx_buf()],
            [idx_buf(), idx_buf()],
            [row_buf(), row_buf()],
            [row_buf(), row_buf()],
            [out_buf(), out_buf()],
            [pltpu.SemaphoreType.DMA] * 2,
            [pltpu.SemaphoreType.DMA] * 2,
            [pltpu.SemaphoreType.DMA] * 2,
            [pltpu.SemaphoreType.DMA] * 2,
        ],
        compiler_params=pltpu.CompilerParams(
            use_tc_tiling_on_sc=False, needs_layout_passes=False),
    )
    def gather_k(idxa_hbm, idxm_hbm, t12_hbm, out_hbm,
                 idxa_v, idxm_v, rows1_v, rows2_v, out_v, isem, gs1, gs2,
                 os):
        wid = lax.axis_index("s") * NUM_SC_CORES + lax.axis_index("c")
        w_base = wid * per_w

        def load_idx(g, si):
            base = w_base + g * chunk
            pltpu.async_copy(idxa_hbm.at[pl.ds(base, chunk)], idxa_v[si],
                             isem[si])
            pltpu.async_copy(idxm_hbm.at[pl.ds(base, chunk)], idxm_v[si],
                             isem[si])

        def wait_idx(g, si):
            base = w_base + g * chunk
            pltpu.make_async_copy(idxa_hbm.at[pl.ds(base, chunk)],
                                  idxa_v[si], isem[si]).wait()
            pltpu.make_async_copy(idxm_hbm.at[pl.ds(base, chunk)],
                                  idxm_v[si], isem[si]).wait()

        def fire(g, s):
            wait_idx(g, s)
            pltpu.async_copy(t12_hbm.at[idxa_v[s]], rows1_v[s], gs1[s])
            pltpu.async_copy(t12_hbm.at[idxm_v[s]], rows2_v[s], gs2[s])

        def handle(g, s):
            @pl.when(g + 1 < n_chunks)
            def _():
                fire(g + 1, s ^ 1)

            pltpu.make_async_copy(t12_hbm.at[idxa_v[s]], rows1_v[s],
                                  gs1[s]).wait()
            pltpu.make_async_copy(t12_hbm.at[idxm_v[s]], rows2_v[s],
                                  gs2[s]).wait()

            @pl.when(g + 2 < n_chunks)
            def _():
                load_idx(g + 2, s)

            @pl.when(g >= 2)
            def _():
                prev = w_base + (g - 2) * chunk
                pltpu.make_async_copy(
                    out_v[s], out_hbm.at[pl.ds(prev, chunk)], os[s]).wait()

            def row_body(j, carry):
                msk = jnp.uint32(0xFFFF0000)
                for c in range(EMB_DIM // LANES):
                    sl = pl.ds(c * LANES, LANES)
                    v1 = rows1_v[s][j, sl]
                    v2 = rows2_v[s][j, sl]
                    lo = (plsc.bitcast(v1 << 16, jnp.float32)
                          + plsc.bitcast(v2 << 16, jnp.float32))
                    hi = (plsc.bitcast(v1 & msk, jnp.float32)
                          + plsc.bitcast(v2 & msk, jnp.float32))
                    out_v[s][j, sl] = jnp.maximum(lo, 0.0)
                    out_v[s][j, pl.ds(EMB_DIM + c * LANES, LANES)] = (
                        jnp.maximum(hi, 0.0))
                return carry

            lax.fori_loop(0, chunk, row_body, 0, unroll=False)
            base = w_base + g * chunk
            pltpu.async_copy(out_v[s], out_hbm.at[pl.ds(base, chunk)],
                             os[s])

        load_idx(0, 0)
        load_idx(1, 1)
        fire(0, 0)

        def body(i, carry):
            handle(2 * i, 0)
            handle(2 * i + 1, 1)
            return carry

        lax.fori_loop(0, n_chunks // 2, body, 0, unroll=False)
        for s in (0, 1):
            last = w_base + (n_chunks - 2 + s) * chunk
            pltpu.make_async_copy(
                out_v[s], out_hbm.at[pl.ds(last, chunk)], os[s]).wait()

    return gather_k(idx_a2, idx_m2, t12)


def kernel(parts, aff_table, mat_table, W, b):
    B, L, _ = parts.shape
    pt = jnp.transpose(parts, (1, 2, 0)).astype(jnp.int32)
    idx_a2 = pt[:, 0, :].reshape(-1) * 2
    idx_m2 = pt[:, 1, :].reshape(-1) * 2 + 1
    w1 = W[:, :EMB_DIM]
    w2 = W[:, EMB_DIM:]
    b2d = b.reshape(1, PART_DIM)
    t12 = _tc_project_packed(aff_table.T, w1, w2, b2d, blk=2048)
    t12v = t12.reshape(2 * t12.shape[0], EMB_DIM)
    out_t = _sc_gather_combine(idx_a2, idx_m2, t12v, chunk=200)
    return jnp.transpose(out_t.reshape(L, B, PART_DIM), (1, 0, 2))


# speedup vs baseline: 1.3483x; 1.3483x over previous
import functools

import jax
import jax.numpy as jnp
from jax import lax
from jax.experimental import pallas as pl
from jax.experimental.pallas import tpu as pltpu
from jax.experimental.pallas import tpu_sc as plsc

EMB_DIM = 64
PART_DIM = 128

NUM_SC_CORES = 2
NUM_SC_SUBCORES = 16
NUM_WORKERS = NUM_SC_CORES * NUM_SC_SUBCORES
LANES = 16


def _tc_project(table_t, w1, w2, b2d, blk):
    vocab = table_t.shape[1]
    nb = (vocab + blk - 1) // blk

    def proj_kernel(t_ref, w1_ref, w2_ref, b_ref, o1_ref, o2_ref):
        tb = t_ref[...]
        acc1 = lax.dot_general(
            tb, w1_ref[...], (((0,), (1,)), ((), ())),
            preferred_element_type=jnp.float32,
        )
        o1_ref[...] = acc1 + b_ref[...]
        o2_ref[...] = lax.dot_general(
            tb, w2_ref[...], (((0,), (1,)), ((), ())),
            preferred_element_type=jnp.float32,
        )

    return pl.pallas_call(
        proj_kernel,
        grid=(nb,),
        in_specs=[
            pl.BlockSpec((EMB_DIM, blk), lambda i: (0, i)),
            pl.BlockSpec((PART_DIM, EMB_DIM), lambda i: (0, 0)),
            pl.BlockSpec((PART_DIM, EMB_DIM), lambda i: (0, 0)),
            pl.BlockSpec((1, PART_DIM), lambda i: (0, 0)),
        ],
        out_specs=[
            pl.BlockSpec((blk, PART_DIM), lambda i: (i, 0)),
            pl.BlockSpec((blk, PART_DIM), lambda i: (i, 0)),
        ],
        out_shape=[
            jax.ShapeDtypeStruct((vocab, PART_DIM), jnp.float32),
            jax.ShapeDtypeStruct((vocab, PART_DIM), jnp.float32),
        ],
    )(table_t, w1, w2, b2d)


def _sc_gather_combine(idx_a, idx_m, t1, t2, chunk):
    n = idx_a.shape[0]
    per_w = n // NUM_WORKERS
    n_chunks = per_w // chunk
    nslots = 2
    ni = 2
    assert per_w % chunk == 0 and chunk % 8 == 0 and n_chunks % nslots == 0

    mesh = plsc.VectorSubcoreMesh(core_axis_name="c", subcore_axis_name="s")

    row_buf = lambda: pltpu.VMEM((chunk, PART_DIM), jnp.float32)
    idx_buf = lambda: pltpu.VMEM((chunk,), jnp.int32)

    @functools.partial(
        pl.kernel,
        mesh=mesh,
        out_type=jax.ShapeDtypeStruct((n, PART_DIM), jnp.float32),
        scratch_types=[
            [idx_buf() for _ in range(ni)],
            [idx_buf() for _ in range(ni)],
            [row_buf() for _ in range(nslots)],
            [row_buf() for _ in range(nslots)],
            [row_buf() for _ in range(nslots)],
            [pltpu.SemaphoreType.DMA] * ni,
            [pltpu.SemaphoreType.DMA] * nslots,
            [pltpu.SemaphoreType.DMA] * nslots,
            [pltpu.SemaphoreType.DMA] * nslots,
        ],
        compiler_params=pltpu.CompilerParams(needs_layout_passes=False),
    )
    def gather_k(idxa_hbm, idxm_hbm, t1_hbm, t2_hbm, out_hbm,
                 idxa_v, idxm_v, rows1_v, rows2_v, out_v, isem, gs1, gs2,
                 os):
        wid = lax.axis_index("s") * NUM_SC_CORES + lax.axis_index("c")
        w_base = wid * per_w

        def load_idx(g, si):
            base = w_base + g * chunk
            pltpu.async_copy(idxa_hbm.at[pl.ds(base, chunk)], idxa_v[si],
                             isem[si])
            pltpu.async_copy(idxm_hbm.at[pl.ds(base, chunk)], idxm_v[si],
                             isem[si])

        def wait_idx(g, si):
            base = w_base + g * chunk
            pltpu.make_async_copy(idxa_hbm.at[pl.ds(base, chunk)],
                                  idxa_v[si], isem[si]).wait()
            pltpu.make_async_copy(idxm_hbm.at[pl.ds(base, chunk)],
                                  idxm_v[si], isem[si]).wait()

        def fire(g, s, si):
            wait_idx(g, si)
            pltpu.async_copy(t1_hbm.at[idxa_v[si]], rows1_v[s], gs1[s])
            pltpu.async_copy(t2_hbm.at[idxm_v[si]], rows2_v[s], gs2[s])

        def handle(g, s):
            @pl.when(g + 1 < n_chunks)
            def _():
                fire(g + 1, s ^ 1, s ^ 1)

            pltpu.make_async_copy(t1_hbm.at[idxa_v[s]], rows1_v[s],
                                  gs1[s]).wait()
            pltpu.make_async_copy(t2_hbm.at[idxm_v[s]], rows2_v[s],
                                  gs2[s]).wait()

            @pl.when(g + 2 < n_chunks)
            def _():
                load_idx(g + 2, s)

            @pl.when(g >= 2)
            def _():
                prev = w_base + (g - 2) * chunk
                pltpu.make_async_copy(
                    out_v[s], out_hbm.at[pl.ds(prev, chunk)], os[s]).wait()

            def row_body(j, carry):
                for c in range(PART_DIM // LANES):
                    sl = pl.ds(c * LANES, LANES)
                    v = rows1_v[s][j, sl] + rows2_v[s][j, sl]
                    out_v[s][j, sl] = jnp.maximum(v, 0.0)
                return carry

            lax.fori_loop(0, chunk, row_body, 0, unroll=False)
            base = w_base + g * chunk
            pltpu.async_copy(out_v[s], out_hbm.at[pl.ds(base, chunk)],
                             os[s])

        load_idx(0, 0)
        load_idx(1, 1)
        fire(0, 0, 0)

        def body(i, carry):
            handle(2 * i, 0)
            handle(2 * i + 1, 1)
            return carry

        lax.fori_loop(0, n_chunks // 2, body, 0, unroll=False)
        for s in (0, 1):
            last = w_base + (n_chunks - 2 + s) * chunk
            pltpu.make_async_copy(
                out_v[s], out_hbm.at[pl.ds(last, chunk)], os[s]).wait()

    return gather_k(idx_a, idx_m, t1, t2)


def kernel(parts, aff_table, mat_table, W, b):
    B, L, _ = parts.shape
    pt = jnp.transpose(parts, (1, 2, 0)).astype(jnp.int32)
    idx_a = pt[:, 0, :].reshape(-1)
    idx_m = pt[:, 1, :].reshape(-1)
    w1 = W[:, :EMB_DIM]
    w2 = W[:, EMB_DIM:]
    t1, t2 = _tc_project(aff_table.T, w1, w2, b.reshape(1, PART_DIM),
                         blk=2048)
    out_t = _sc_gather_combine(idx_a, idx_m, t1, t2, chunk=160)
    return jnp.transpose(out_t.reshape(L, B, PART_DIM), (1, 0, 2))
